# baseline (speedup 1.0000x reference)
import functools

import jax
import jax.numpy as jnp
from jax.experimental import pallas as pl


def _conv_stats_kernel(x_ref, w_ref, h_ref, s_ref, *, rb, n_s, co):
    b = pl.program_id(0)
    r = pl.program_id(1)

    @pl.when((b == 0) & (r == 0))
    def _init():
        s_ref[...] = jnp.zeros_like(s_ref)

    acc = jnp.zeros((rb * n_s, co), jnp.float32)
    for ky in range(3):
        base = (r * rb + ky) * n_s
        acc += jnp.dot(x_ref[0, pl.ds(base, rb * n_s), :], w_ref[ky],
                       preferred_element_type=jnp.float32)
    s_ref[...] += jnp.concatenate(
        [jnp.sum(acc, axis=0, keepdims=True),
         jnp.sum(acc * acc, axis=0, keepdims=True)], axis=0)
    h_ref[0] = acc.astype(jnp.bfloat16)


def _bn_proj_kernel(h_ref, s_ref, gb_ref, w2_ref, b2_ref, o_ref, *,
                    n_total, eps):
    inv_n = 1.0 / n_total
    mean = s_ref[0:1, :] * inv_n
    var = s_ref[1:2, :] * inv_n - mean * mean
    rstd = jax.lax.rsqrt(var + eps)
    scale = (gb_ref[0:1, :] * rstd).astype(jnp.bfloat16)
    shift = (gb_ref[1:2, :] - mean * gb_ref[0:1, :] * rstd)
    shift = shift.astype(jnp.bfloat16)
    z = h_ref[...] * scale + shift
    y = jnp.maximum(z, jnp.bfloat16(0.1) * z)
    o_ref[...] = (jnp.dot(y, w2_ref[...], preferred_element_type=jnp.float32)
                  + b2_ref[...])


def _xprep_kernel(x_ref, o_ref, *, n_s, ic):
    s2 = n_s * n_s
    wr = (n_s + 2) * n_s
    vt = jnp.transpose(x_ref[0]).astype(jnp.bfloat16)
    row = jax.lax.broadcasted_iota(jnp.int32, (wr, 1), 0) % n_s
    for kx in range(3):
        shift = n_s + 1 - kx
        col = jnp.concatenate(
            [jnp.zeros((shift, ic), jnp.bfloat16), vt,
             jnp.zeros((n_s - 1 + kx, ic), jnp.bfloat16)], axis=0)
        if kx == 0:
            col = jnp.where(row != 0, col, jnp.bfloat16(0))
        elif kx == 2:
            col = jnp.where(row != n_s - 1, col, jnp.bfloat16(0))
        o_ref[0, :, kx * ic:(kx + 1) * ic] = col


def _wprep_kernel(w_ref, o_ref, *, ic):
    vt = jnp.transpose(w_ref[...].astype(jnp.bfloat16))
    v3 = vt.reshape(ic, 9, vt.shape[1])
    for ky in range(3):
        for kx in range(3):
            o_ref[ky, kx * ic:(kx + 1) * ic, :] = v3[:, ky * 3 + kx, :]


def _w1_relayout(w1):
    co, ic = w1.shape[0], w1.shape[1]
    cob = min(co, 512 if ic <= 512 else 256)
    return pl.pallas_call(
        functools.partial(_wprep_kernel, ic=ic),
        grid=(co // cob,),
        in_specs=[pl.BlockSpec((cob, ic * 9), lambda j: (j, 0))],
        out_specs=pl.BlockSpec((3, 3 * ic, cob), lambda j: (0, 0, j)),
        out_shape=jax.ShapeDtypeStruct((3, 3 * ic, co), jnp.bfloat16),
    )(w1.reshape(co, ic * 9))


def _pass1(x, p, rb):
    B, ic, S, _ = x.shape
    co = p['w1'].shape[0]

    xf = pl.pallas_call(
        functools.partial(_xprep_kernel, n_s=S, ic=ic),
        grid=(B,),
        in_specs=[pl.BlockSpec((1, ic, S * S), lambda b: (b, 0, 0))],
        out_specs=pl.BlockSpec((1, (S + 2) * S, 3 * ic), lambda b: (b, 0, 0)),
        out_shape=jax.ShapeDtypeStruct((B, (S + 2) * S, 3 * ic),
                                       jnp.bfloat16),
    )(x.reshape(B, ic, S * S))
    w1t = _w1_relayout(p['w1'])

    nrb = S // rb
    return pl.pallas_call(
        functools.partial(_conv_stats_kernel, rb=rb, n_s=S, co=co),
        grid=(B, nrb),
        in_specs=[
            pl.BlockSpec((1, (S + 2) * S, 3 * ic), lambda b, r: (b, 0, 0)),
            pl.BlockSpec((3, 3 * ic, co), lambda b, r: (0, 0, 0)),
        ],
        out_specs=[
            pl.BlockSpec((1, rb * S, co), lambda b, r: (b, r, 0)),
            pl.BlockSpec((2, co), lambda b, r: (0, 0)),
        ],
        out_shape=[
            jax.ShapeDtypeStruct((B, S * S, co), jnp.bfloat16),
            jax.ShapeDtypeStruct((2, co), jnp.float32),
        ],
    )(xf, w1t)


def _pass2(h1, stats, p, n_total, mb):
    B, ss, co = h1.shape
    no = p['w2'].shape[0]
    nop = ((no + 127) // 128) * 128

    M = B * ss
    h1f = h1.reshape(M, co)
    gb = jnp.stack([p['g'], p['b']], axis=0).astype(jnp.float32)
    w2t = jnp.pad(p['w2'].reshape(no, co).T, ((0, 0), (0, nop - no)))
    w2t = w2t.astype(jnp.bfloat16)
    b2p = jnp.pad(p['b2'], (0, nop - no)).reshape(1, nop).astype(jnp.float32)

    out = pl.pallas_call(
        functools.partial(_bn_proj_kernel, n_total=float(n_total), eps=1e-5),
        grid=(M // mb,),
        in_specs=[
            pl.BlockSpec((mb, co), lambda i: (i, 0)),
            pl.BlockSpec((2, co), lambda i: (0, 0)),
            pl.BlockSpec((2, co), lambda i: (0, 0)),
            pl.BlockSpec((co, nop), lambda i: (0, 0)),
            pl.BlockSpec((1, nop), lambda i: (0, 0)),
        ],
        out_specs=pl.BlockSpec((mb, nop), lambda i: (i, 0)),
        out_shape=jax.ShapeDtypeStruct((M, nop), jnp.float32),
    )(h1f, stats, gb, w2t, b2p)
    return out


_SCALE_CFG = ((32, 2048), (32, 2048), (16, 1024))


def kernel(feat0, feat1, feat2, params):
    outs = []
    for x, p, (rb, mb) in zip((feat0, feat1, feat2), params, _SCALE_CFG):
        B, _, S, _ = x.shape
        no = p['w2'].shape[0]
        h1, stats = _pass1(x, p, rb)
        out = _pass2(h1, stats, p, n_total=B * S * S, mb=mb)
        nop = out.shape[-1]
        outs.append(out.reshape(B, S, S, nop)[..., :no])
    return tuple(outs)


# speedup vs baseline: 1.5496x; 1.5496x over previous
import functools

import jax
import jax.numpy as jnp
from jax.experimental import pallas as pl


def _conv_stats_kernel(x_ref, w_ref, h_ref, s_ref, *, rb, n_s, co):
    b = pl.program_id(0)
    r = pl.program_id(1)

    @pl.when((b == 0) & (r == 0))
    def _init():
        s_ref[...] = jnp.zeros_like(s_ref)

    acc = jnp.zeros((rb * n_s, co), jnp.float32)
    for ky in range(3):
        base = (r * rb + ky) * n_s
        acc += jnp.dot(x_ref[0, pl.ds(base, rb * n_s), :], w_ref[ky],
                       preferred_element_type=jnp.float32)
    s_ref[...] += jnp.concatenate(
        [jnp.sum(acc, axis=0, keepdims=True),
         jnp.sum(acc * acc, axis=0, keepdims=True)], axis=0)
    h_ref[0] = acc.astype(jnp.bfloat16)


def _bn_proj_kernel(h_ref, s_ref, gb_ref, w2_ref, b2_ref, o_ref, *,
                    n_total, eps):
    inv_n = 1.0 / n_total
    mean = s_ref[0:1, :] * inv_n
    var = s_ref[1:2, :] * inv_n - mean * mean
    rstd = jax.lax.rsqrt(var + eps)
    scale = (gb_ref[0:1, :] * rstd).astype(jnp.bfloat16)
    shift = (gb_ref[1:2, :] - mean * gb_ref[0:1, :] * rstd)
    shift = shift.astype(jnp.bfloat16)
    z = h_ref[...] * scale + shift
    y = jnp.maximum(z, jnp.bfloat16(0.1) * z)
    o_ref[...] = (jnp.dot(y, w2_ref[...], preferred_element_type=jnp.float32)
                  + b2_ref[...])


def _xprep_kernel(x_ref, o_ref, *, n_s, ic):
    s2 = n_s * n_s
    wr = (n_s + 2) * n_s
    vt = jnp.transpose(x_ref[0]).astype(jnp.bfloat16)
    row = jax.lax.broadcasted_iota(jnp.int32, (wr, 1), 0) % n_s
    for kx in range(3):
        shift = n_s + 1 - kx
        col = jnp.concatenate(
            [jnp.zeros((shift, ic), jnp.bfloat16), vt,
             jnp.zeros((n_s - 1 + kx, ic), jnp.bfloat16)], axis=0)
        if kx == 0:
            col = jnp.where(row != 0, col, jnp.bfloat16(0))
        elif kx == 2:
            col = jnp.where(row != n_s - 1, col, jnp.bfloat16(0))
        o_ref[0, :, kx * ic:(kx + 1) * ic] = col


def _w1_relayout(w1):
    co, ic = w1.shape[0], w1.shape[1]
    w1t = jnp.transpose(w1, (2, 3, 1, 0)).reshape(3, 3 * ic, co)
    return w1t.astype(jnp.bfloat16)


def _pass1(x, p, rb):
    B, ic, S, _ = x.shape
    co = p['w1'].shape[0]

    xf = pl.pallas_call(
        functools.partial(_xprep_kernel, n_s=S, ic=ic),
        grid=(B,),
        in_specs=[pl.BlockSpec((1, ic, S * S), lambda b: (b, 0, 0))],
        out_specs=pl.BlockSpec((1, (S + 2) * S, 3 * ic), lambda b: (b, 0, 0)),
        out_shape=jax.ShapeDtypeStruct((B, (S + 2) * S, 3 * ic),
                                       jnp.bfloat16),
    )(x.reshape(B, ic, S * S))
    w1t = _w1_relayout(p['w1'])

    nrb = S // rb
    return pl.pallas_call(
        functools.partial(_conv_stats_kernel, rb=rb, n_s=S, co=co),
        grid=(B, nrb),
        in_specs=[
            pl.BlockSpec((1, (S + 2) * S, 3 * ic), lambda b, r: (b, 0, 0)),
            pl.BlockSpec((3, 3 * ic, co), lambda b, r: (0, 0, 0)),
        ],
        out_specs=[
            pl.BlockSpec((1, rb * S, co), lambda b, r: (b, r, 0)),
            pl.BlockSpec((2, co), lambda b, r: (0, 0)),
        ],
        out_shape=[
            jax.ShapeDtypeStruct((B, S * S, co), jnp.bfloat16),
            jax.ShapeDtypeStruct((2, co), jnp.float32),
        ],
    )(xf, w1t)


def _pass2(h1, stats, p, n_total, mb):
    B, ss, co = h1.shape
    no = p['w2'].shape[0]
    nop = ((no + 127) // 128) * 128

    M = B * ss
    h1f = h1.reshape(M, co)
    gb = jnp.stack([p['g'], p['b']], axis=0).astype(jnp.float32)
    w2t = jnp.pad(p['w2'].reshape(no, co).T, ((0, 0), (0, nop - no)))
    w2t = w2t.astype(jnp.bfloat16)
    b2p = jnp.pad(p['b2'], (0, nop - no)).reshape(1, nop).astype(jnp.float32)

    out = pl.pallas_call(
        functools.partial(_bn_proj_kernel, n_total=float(n_total), eps=1e-5),
        grid=(M // mb,),
        in_specs=[
            pl.BlockSpec((mb, co), lambda i: (i, 0)),
            pl.BlockSpec((2, co), lambda i: (0, 0)),
            pl.BlockSpec((2, co), lambda i: (0, 0)),
            pl.BlockSpec((co, nop), lambda i: (0, 0)),
            pl.BlockSpec((1, nop), lambda i: (0, 0)),
        ],
        out_specs=pl.BlockSpec((mb, nop), lambda i: (i, 0)),
        out_shape=jax.ShapeDtypeStruct((M, nop), jnp.float32),
    )(h1f, stats, gb, w2t, b2p)
    return out


_SCALE_CFG = ((32, 2048), (32, 2048), (16, 1024))


def kernel(feat0, feat1, feat2, params):
    outs = []
    for x, p, (rb, mb) in zip((feat0, feat1, feat2), params, _SCALE_CFG):
        B, _, S, _ = x.shape
        no = p['w2'].shape[0]
        h1, stats = _pass1(x, p, rb)
        out = _pass2(h1, stats, p, n_total=B * S * S, mb=mb)
        nop = out.shape[-1]
        outs.append(out.reshape(B, S, S, nop)[..., :no])
    return tuple(outs)
